# baseline (speedup 1.0000x reference)
import functools

import jax
import jax.numpy as jnp
from jax import lax
from jax.experimental import pallas as pl
from jax.experimental.pallas import tpu as pltpu
from jax.experimental.pallas import tpu_sc as plsc

_NC = 2
_NS = 16
_NW = _NC * _NS

_LEVELS = [40962, 10242, 2562, 642, 162]
_NLEV = 5


def _gather_plan(B, D, group):
    m = {7: 56, 2: 16, 1: 16}[group]
    limit = 500_000
    best = None
    mult = 1
    while True:
        cr = m * mult
        bufb = cr * D * 4
        if bufb * 2 > limit and mult > 1:
            break
        crp = -(-cr // 16) * 16
        for nbuf in (3, 2, 1):
            k0 = -(-B // (_NW * cr))
            k = -(-k0 // nbuf) * nbuf
            if nbuf * bufb + k * crp * 4 > limit:
                continue
            bp = _NW * cr * k
            score = bp * D * 4 * (1.0 if nbuf > 1 else 1.8) + k * 60_000
            if best is None or score < best[0]:
                best = (score, cr, crp, k, nbuf, bp)
        mult += 1
    _, cr, crp, k, nbuf, bp = best
    return cr, crp, k, nbuf, bp


def _sc_gather(table, idx, group):
    V, D = table.shape
    B = idx.shape[0]
    cr, crp, k, nbuf, bp = _gather_plan(B, D, group)
    idx_p = jnp.pad(idx, (0, bp - B))
    idx3 = idx_p.reshape(_NW, k, cr)
    if crp != cr:
        idx3 = jnp.pad(idx3, ((0, 0), (0, 0), (0, crp - cr)))
    mesh = plsc.VectorSubcoreMesh(core_axis_name="c", subcore_axis_name="s")
    nb = min(nbuf, k)
    lookahead = nb - 1

    @functools.partial(
        pl.kernel,
        mesh=mesh,
        compiler_params=pltpu.CompilerParams(use_tc_tiling_on_sc=False),
        out_type=jax.ShapeDtypeStruct((bp, D), jnp.float32),
        scratch_types=[
            pltpu.VMEM((k, crp), jnp.int32),
            [pltpu.VMEM((cr, D), jnp.float32) for _ in range(nb)],
            [pltpu.SemaphoreType.DMA for _ in range(nb)],
            [pltpu.SemaphoreType.DMA for _ in range(nb)],
        ],
    )
    def gk(table_hbm, idx_hbm, out_hbm, idx_v, bufs, gsem, ssem):
        wid = lax.axis_index("s") * _NC + lax.axis_index("c")
        pltpu.sync_copy(idx_hbm.at[wid], idx_v)
        base = wid * (k * cr)

        def start_gather(c, b):
            pltpu.async_copy(
                table_hbm.at[idx_v.at[c, pl.ds(0, cr)]], bufs[b], gsem[b])

        def wait_gather(b):
            pltpu.make_async_copy(
                table_hbm.at[idx_v.at[0, pl.ds(0, cr)]], bufs[b], gsem[b]).wait()

        def start_scatter(c, b):
            pltpu.async_copy(
                bufs[b], out_hbm.at[pl.ds(base + c * cr, cr)], ssem[b])

        def wait_scatter(b):
            pltpu.make_async_copy(
                bufs[b], out_hbm.at[pl.ds(base, cr)], ssem[b]).wait()

        if nb == 1:
            def body(c, carry):
                pltpu.async_copy(
                    table_hbm.at[idx_v.at[c, pl.ds(0, cr)]], bufs[0],
                    gsem[0])
                wait_gather(0)
                pltpu.sync_copy(bufs[0], out_hbm.at[pl.ds(base + c * cr, cr)])
                return carry

            lax.fori_loop(0, k, body, 0)
            return

        for b0 in range(lookahead):
            start_gather(b0, b0)

        def body(g, carry):
            for b in range(nb):
                c = g * nb + b
                wait_gather(b)
                start_scatter(c, b)
                b2 = (b + lookahead) % nb

                @pl.when(c + lookahead < k)
                def _():
                    @pl.when(c + lookahead - nb >= 0)
                    def _():
                        wait_scatter(b2)

                    start_gather(c + lookahead, b2)

            return carry

        lax.fori_loop(0, k // nb, body, 0)
        for b in range(nb):
            wait_scatter(b)

    return gk(table, idx3)


def _mm(G, W, b, n_true=None):
    Np, K = G.shape
    oc = W.shape[0]
    bn = min(256, Np)
    grid = Np // bn
    b2 = b.reshape(1, oc)
    stats = n_true is not None

    def kern(g_ref, w_ref, b_ref, y_ref, *rest):
        y = lax.dot_general(
            g_ref[...], w_ref[...], (((1,), (1,)), ((), ())),
            preferred_element_type=jnp.float32) + b_ref[...]
        y_ref[...] = y
        if stats:
            s_ref = rest[0]
            i = pl.program_id(0)
            rows = i * bn + lax.broadcasted_iota(jnp.int32, (bn, 1), 0)
            ym = jnp.where(rows < n_true, y, 0.0)
            contrib = jnp.concatenate(
                [jnp.sum(ym, axis=0, keepdims=True),
                 jnp.sum(ym * ym, axis=0, keepdims=True),
                 jnp.zeros((6, oc), jnp.float32)], axis=0)

            @pl.when(i == 0)
            def _():
                s_ref[...] = jnp.zeros_like(s_ref)

            s_ref[...] += contrib

    out_shape = [jax.ShapeDtypeStruct((Np, oc), jnp.float32)]
    out_specs = [pl.BlockSpec((bn, oc), lambda i: (i, 0))]
    if stats:
        out_shape.append(jax.ShapeDtypeStruct((8, oc), jnp.float32))
        out_specs.append(pl.BlockSpec((8, oc), lambda i: (0, 0)))
    res = pl.pallas_call(
        kern,
        grid=(grid,),
        in_specs=[
            pl.BlockSpec((bn, K), lambda i: (i, 0)),
            pl.BlockSpec((oc, K), lambda i: (0, 0)),
            pl.BlockSpec((1, oc), lambda i: (0, 0)),
        ],
        out_specs=out_specs,
        out_shape=out_shape,
    )(G, W, b2)
    return res if stats else res[0]


def _bn_lrelu(Y, S, gamma, beta, n_true):
    Np, oc = Y.shape
    bn = min(256, Np)
    grid = Np // bn
    g2 = gamma.reshape(1, oc)
    be2 = beta.reshape(1, oc)
    inv_n = 1.0 / float(n_true)

    def kern(y_ref, s_ref, g_ref, be_ref, o_ref):
        m = s_ref[0:1, :] * inv_n
        var = s_ref[1:2, :] * inv_n - m * m
        scale = g_ref[...] * lax.rsqrt(var + 1e-5)
        shift = be_ref[...] - m * scale
        h = y_ref[...] * scale + shift
        o_ref[...] = jnp.where(h >= 0, h, 0.2 * h)

    return pl.pallas_call(
        kern,
        grid=(grid,),
        in_specs=[
            pl.BlockSpec((bn, oc), lambda i: (i, 0)),
            pl.BlockSpec((8, oc), lambda i: (0, 0)),
            pl.BlockSpec((1, oc), lambda i: (0, 0)),
            pl.BlockSpec((1, oc), lambda i: (0, 0)),
        ],
        out_specs=pl.BlockSpec((bn, oc), lambda i: (i, 0)),
        out_shape=jax.ShapeDtypeStruct((Np, oc), jnp.float32),
    )(Y, S, g2, be2)


def _group_sum_stats(G, g, b, n_true):
    Bp, oc = G.shape
    rows = Bp // g
    bg = min(256, rows)
    grid = rows // bg
    b2 = b.reshape(1, oc)

    def kern(g_ref, b_ref, y_ref, s_ref):
        i = pl.program_id(0)
        y = jnp.sum(g_ref[...].reshape(bg, g, oc), axis=1) + b_ref[...]
        y_ref[...] = y
        r = i * bg + lax.broadcasted_iota(jnp.int32, (bg, 1), 0)
        ym = jnp.where(r < n_true, y, 0.0)
        contrib = jnp.concatenate(
            [jnp.sum(ym, axis=0, keepdims=True),
             jnp.sum(ym * ym, axis=0, keepdims=True),
             jnp.zeros((6, oc), jnp.float32)], axis=0)

        @pl.when(i == 0)
        def _():
            s_ref[...] = jnp.zeros_like(s_ref)

        s_ref[...] += contrib

    return pl.pallas_call(
        kern,
        grid=(grid,),
        in_specs=[
            pl.BlockSpec((g * bg, oc), lambda i: (i, 0)),
            pl.BlockSpec((1, oc), lambda i: (0, 0)),
        ],
        out_specs=[
            pl.BlockSpec((bg, oc), lambda i: (i, 0)),
            pl.BlockSpec((8, oc), lambda i: (0, 0)),
        ],
        out_shape=[
            jax.ShapeDtypeStruct((rows, oc), jnp.float32),
            jax.ShapeDtypeStruct((8, oc), jnp.float32),
        ],
    )(G, b2)


def _group_mean(G, g):
    Bp, D = G.shape
    rows = Bp // g
    bg = min(256, rows)
    grid = rows // bg

    def kern(g_ref, o_ref):
        xb = g_ref[...].reshape(bg, g, D)
        o_ref[...] = jnp.mean(xb, axis=1)

    return pl.pallas_call(
        kern,
        grid=(grid,),
        in_specs=[pl.BlockSpec((g * bg, D), lambda i: (i, 0))],
        out_specs=pl.BlockSpec((bg, D), lambda i: (i, 0)),
        out_shape=jax.ShapeDtypeStruct((rows, D), jnp.float32),
    )(G)


def _conv_block(h, neigh, p, n_true):
    ic = h.shape[1]
    G = _sc_gather(h, neigh, 7)
    G = G.reshape(G.shape[0] // 7, 7 * ic)
    Y, S = _mm(G, p['W1'], p['b1'], n_true=n_true)
    h = _bn_lrelu(Y, S, p['g1'], p['be1'], n_true)
    oc = h.shape[1]
    G = _sc_gather(h, neigh, 7)
    G = G.reshape(G.shape[0] // 7, 7 * oc)
    Y, S = _mm(G, p['W2'], p['b2'], n_true=n_true)
    return _bn_lrelu(Y, S, p['g2'], p['be2'], n_true)


def kernel(x, params, neighs, up_top, up_down):
    skips = [None] * _NLEV
    h = x
    for i in range(_NLEV):
        n_i = _LEVELS[i]
        if i > 0:
            pool_idx = neighs[i - 1][: 7 * n_i]
            P = _sc_gather(h, pool_idx, 7)
            h = _group_mean(P, 7)
        h = _conv_block(h, neighs[i], params['down'][i], n_i)
        skips[i] = h
    for i in range(_NLEV - 1):
        p = params['up'][i]
        lvl = _NLEV - 2 - i
        n_c = _LEVELS[lvl + 1]
        n_f = _LEVELS[lvl]
        oc = p['W2'].shape[0]
        y = _mm(h, p['Wu'], p['bu'])
        yv = y.reshape(y.shape[0] * 7, oc)
        ti2 = jnp.repeat(up_top[i], 2)
        idx2 = jnp.concatenate([ti2, up_down[i]])
        G2 = _sc_gather(yv, idx2, 2)
        u = _group_mean(G2, 2)
        npad = -(-n_f // 256) * 256
        table = jnp.concatenate([u[:npad], skips[lvl][:npad]], axis=1)
        ic = table.shape[1]
        W1r = p['W1'].reshape(oc, 7, ic).transpose(1, 0, 2).reshape(7 * oc, ic)
        A = _mm(table, W1r, jnp.zeros((7 * oc,), jnp.float32))
        A2 = A.reshape(A.shape[0] * 7, oc)
        idx7 = neighs[lvl] * 7 + jnp.tile(jnp.arange(7, dtype=jnp.int32), n_f)
        G7 = _sc_gather(A2, idx7, 7)
        Y, S = _group_sum_stats(G7, 7, p['b1'], n_f)
        h1 = _bn_lrelu(Y, S, p['g1'], p['be1'], n_f)
        G = _sc_gather(h1, neighs[lvl], 7)
        G = G.reshape(G.shape[0] // 7, 7 * oc)
        Y, S = _mm(G, p['W2'], p['b2'], n_true=n_f)
        h = _bn_lrelu(Y, S, p['g2'], p['be2'], n_f)
    out = _mm(h, params['outW'], params['outb'])
    return out[: _LEVELS[0]]


# speedup vs baseline: 1.1217x; 1.1217x over previous
import functools

import jax
import jax.numpy as jnp
from jax import lax
from jax.experimental import pallas as pl
from jax.experimental.pallas import tpu as pltpu
from jax.experimental.pallas import tpu_sc as plsc

_NC = 2
_NS = 16
_NW = _NC * _NS

_LEVELS = [40962, 10242, 2562, 642, 162]
_NLEV = 5


def _gather_plan(B, D, group, exact=False):
    m = {7: 56, 2: 16, 1: 16}[group]
    limit = 500_000
    best = None
    mult = 1
    while True:
        cr = m * mult
        bufb = cr * D * 4
        if bufb * 2 > limit and mult > 1:
            break
        crp = -(-cr // 16) * 16
        for nbuf in (3, 2, 1):
            k0 = -(-B // (_NW * cr))
            k = -(-k0 // nbuf) * nbuf
            if nbuf * bufb + k * crp * 4 > limit:
                continue
            bp = _NW * cr * k
            if exact and bp != B:
                continue
            score = bp * D * 4 * (1.0 if nbuf > 1 else 1.8) + k * 60_000
            if best is None or score < best[0]:
                best = (score, cr, crp, k, nbuf, bp)
        mult += 1
    _, cr, crp, k, nbuf, bp = best
    return cr, crp, k, nbuf, bp


def _sc_gather(table, idx, group, exact=False):
    V, D = table.shape
    B = idx.shape[0]
    cr, crp, k, nbuf, bp = _gather_plan(B, D, group, exact)
    idx_p = jnp.pad(idx, (0, bp - B))
    idx3 = idx_p.reshape(_NW, k, cr)
    if crp != cr:
        idx3 = jnp.pad(idx3, ((0, 0), (0, 0), (0, crp - cr)))
    mesh = plsc.VectorSubcoreMesh(core_axis_name="c", subcore_axis_name="s")
    nb = min(nbuf, k)
    lookahead = nb - 1

    @functools.partial(
        pl.kernel,
        mesh=mesh,
        compiler_params=pltpu.CompilerParams(use_tc_tiling_on_sc=False),
        out_type=jax.ShapeDtypeStruct((bp, D), jnp.float32),
        scratch_types=[
            pltpu.VMEM((k, crp), jnp.int32),
            [pltpu.VMEM((cr, D), jnp.float32) for _ in range(nb)],
            [pltpu.SemaphoreType.DMA for _ in range(nb)],
            [pltpu.SemaphoreType.DMA for _ in range(nb)],
        ],
    )
    def gk(table_hbm, idx_hbm, out_hbm, idx_v, bufs, gsem, ssem):
        wid = lax.axis_index("s") * _NC + lax.axis_index("c")
        pltpu.sync_copy(idx_hbm.at[wid], idx_v)
        base = wid * (k * cr)

        def start_gather(c, b):
            pltpu.async_copy(
                table_hbm.at[idx_v.at[c, pl.ds(0, cr)]], bufs[b], gsem[b])

        def wait_gather(b):
            pltpu.make_async_copy(
                table_hbm.at[idx_v.at[0, pl.ds(0, cr)]], bufs[b], gsem[b]).wait()

        def start_scatter(c, b):
            pltpu.async_copy(
                bufs[b], out_hbm.at[pl.ds(base + c * cr, cr)], ssem[b])

        def wait_scatter(b):
            pltpu.make_async_copy(
                bufs[b], out_hbm.at[pl.ds(base, cr)], ssem[b]).wait()

        if nb == 1:
            def body(c, carry):
                pltpu.async_copy(
                    table_hbm.at[idx_v.at[c, pl.ds(0, cr)]], bufs[0],
                    gsem[0])
                wait_gather(0)
                pltpu.sync_copy(bufs[0], out_hbm.at[pl.ds(base + c * cr, cr)])
                return carry

            lax.fori_loop(0, k, body, 0)
            return

        for b0 in range(lookahead):
            start_gather(b0, b0)

        def body(g, carry):
            for b in range(nb):
                c = g * nb + b
                wait_gather(b)
                start_scatter(c, b)
                b2 = (b + lookahead) % nb

                @pl.when(c + lookahead < k)
                def _():
                    @pl.when(c + lookahead - nb >= 0)
                    def _():
                        wait_scatter(b2)

                    start_gather(c + lookahead, b2)

            return carry

        lax.fori_loop(0, k // nb, body, 0)
        for b in range(nb):
            wait_scatter(b)

    return gk(table, idx3)


def _mm(G, W, b, n_true=None):
    Np, K = G.shape
    oc = W.shape[0]
    bn = min(256, Np)
    grid = Np // bn
    b2 = b.reshape(1, oc)
    stats = n_true is not None

    def kern(g_ref, w_ref, b_ref, y_ref, *rest):
        y = lax.dot_general(
            g_ref[...], w_ref[...], (((1,), (1,)), ((), ())),
            preferred_element_type=jnp.float32) + b_ref[...]
        y_ref[...] = y
        if stats:
            s_ref = rest[0]
            i = pl.program_id(0)
            rows = i * bn + lax.broadcasted_iota(jnp.int32, (bn, 1), 0)
            ym = jnp.where(rows < n_true, y, 0.0)
            contrib = jnp.concatenate(
                [jnp.sum(ym, axis=0, keepdims=True),
                 jnp.sum(ym * ym, axis=0, keepdims=True),
                 jnp.zeros((6, oc), jnp.float32)], axis=0)

            @pl.when(i == 0)
            def _():
                s_ref[...] = jnp.zeros_like(s_ref)

            s_ref[...] += contrib

    out_shape = [jax.ShapeDtypeStruct((Np, oc), jnp.float32)]
    out_specs = [pl.BlockSpec((bn, oc), lambda i: (i, 0))]
    if stats:
        out_shape.append(jax.ShapeDtypeStruct((8, oc), jnp.float32))
        out_specs.append(pl.BlockSpec((8, oc), lambda i: (0, 0)))
    res = pl.pallas_call(
        kern,
        grid=(grid,),
        in_specs=[
            pl.BlockSpec((bn, K), lambda i: (i, 0)),
            pl.BlockSpec((oc, K), lambda i: (0, 0)),
            pl.BlockSpec((1, oc), lambda i: (0, 0)),
        ],
        out_specs=out_specs,
        out_shape=out_shape,
    )(G, W, b2)
    return res if stats else res[0]


def _bn_lrelu(Y, S, gamma, beta, n_true):
    Np, oc = Y.shape
    bn = min(256, Np)
    grid = Np // bn
    g2 = gamma.reshape(1, oc)
    be2 = beta.reshape(1, oc)
    inv_n = 1.0 / float(n_true)

    def kern(y_ref, s_ref, g_ref, be_ref, o_ref):
        m = s_ref[0:1, :] * inv_n
        var = s_ref[1:2, :] * inv_n - m * m
        scale = g_ref[...] * lax.rsqrt(var + 1e-5)
        shift = be_ref[...] - m * scale
        h = y_ref[...] * scale + shift
        o_ref[...] = jnp.where(h >= 0, h, 0.2 * h)

    return pl.pallas_call(
        kern,
        grid=(grid,),
        in_specs=[
            pl.BlockSpec((bn, oc), lambda i: (i, 0)),
            pl.BlockSpec((8, oc), lambda i: (0, 0)),
            pl.BlockSpec((1, oc), lambda i: (0, 0)),
            pl.BlockSpec((1, oc), lambda i: (0, 0)),
        ],
        out_specs=pl.BlockSpec((bn, oc), lambda i: (i, 0)),
        out_shape=jax.ShapeDtypeStruct((Np, oc), jnp.float32),
    )(Y, S, g2, be2)


def _bn_lrelu2(YA, YB, SA, SB, gamma, beta, n_tot):
    NpA, oc = YA.shape
    NpB = YB.shape[0]
    bn = 256
    gA = NpA // bn
    gB = NpB // bn
    g2 = gamma.reshape(1, oc)
    be2 = beta.reshape(1, oc)
    inv_n = 1.0 / float(n_tot)

    def kern(ya_ref, yb_ref, sa_ref, sb_ref, g_ref, be_ref, o_ref):
        i = pl.program_id(0)
        s0 = sa_ref[0:1, :] + sb_ref[0:1, :]
        s1 = sa_ref[1:2, :] + sb_ref[1:2, :]
        m = s0 * inv_n
        var = s1 * inv_n - m * m
        scale = g_ref[...] * lax.rsqrt(var + 1e-5)
        shift = be_ref[...] - m * scale
        y = jnp.where(i < gA, ya_ref[...], yb_ref[...])
        h = y * scale + shift
        o_ref[...] = jnp.where(h >= 0, h, 0.2 * h)

    return pl.pallas_call(
        kern,
        grid=(gA + gB,),
        in_specs=[
            pl.BlockSpec((bn, oc), lambda i: (jnp.minimum(i, gA - 1), 0)),
            pl.BlockSpec((bn, oc), lambda i: (jnp.maximum(i - gA, 0), 0)),
            pl.BlockSpec((8, oc), lambda i: (0, 0)),
            pl.BlockSpec((8, oc), lambda i: (0, 0)),
            pl.BlockSpec((1, oc), lambda i: (0, 0)),
            pl.BlockSpec((1, oc), lambda i: (0, 0)),
        ],
        out_specs=pl.BlockSpec((bn, oc), lambda i: (i, 0)),
        out_shape=jax.ShapeDtypeStruct((NpA + NpB, oc), jnp.float32),
    )(YA, YB, SA, SB, g2, be2)


def _group_sum_stats(G, g, b, n_true):
    Bp, oc = G.shape
    rows = Bp // g
    bg = min(256, rows)
    grid = rows // bg
    b2 = b.reshape(1, oc)

    def kern(g_ref, b_ref, y_ref, s_ref):
        i = pl.program_id(0)
        y = jnp.sum(g_ref[...].reshape(bg, g, oc), axis=1) + b_ref[...]
        y_ref[...] = y
        r = i * bg + lax.broadcasted_iota(jnp.int32, (bg, 1), 0)
        ym = jnp.where(r < n_true, y, 0.0)
        contrib = jnp.concatenate(
            [jnp.sum(ym, axis=0, keepdims=True),
             jnp.sum(ym * ym, axis=0, keepdims=True),
             jnp.zeros((6, oc), jnp.float32)], axis=0)

        @pl.when(i == 0)
        def _():
            s_ref[...] = jnp.zeros_like(s_ref)

        s_ref[...] += contrib

    return pl.pallas_call(
        kern,
        grid=(grid,),
        in_specs=[
            pl.BlockSpec((g * bg, oc), lambda i: (i, 0)),
            pl.BlockSpec((1, oc), lambda i: (0, 0)),
        ],
        out_specs=[
            pl.BlockSpec((bg, oc), lambda i: (i, 0)),
            pl.BlockSpec((8, oc), lambda i: (0, 0)),
        ],
        out_shape=[
            jax.ShapeDtypeStruct((rows, oc), jnp.float32),
            jax.ShapeDtypeStruct((8, oc), jnp.float32),
        ],
    )(G, b2)


def _group_mean(G, g):
    Bp, D = G.shape
    rows = Bp // g
    bg = min(256, rows)
    grid = rows // bg

    def kern(g_ref, o_ref):
        xb = g_ref[...].reshape(bg, g, D)
        o_ref[...] = jnp.mean(xb, axis=1)

    return pl.pallas_call(
        kern,
        grid=(grid,),
        in_specs=[pl.BlockSpec((g * bg, D), lambda i: (i, 0))],
        out_specs=pl.BlockSpec((bg, D), lambda i: (i, 0)),
        out_shape=jax.ShapeDtypeStruct((rows, D), jnp.float32),
    )(G)


_SPLIT_MIN = 10242


def _conv_layer(h, neigh, W, b, gam, bet, n):
    ic = h.shape[1]
    if n >= _SPLIT_MIN:
        sA = 256 * (n // 512)
        GA = _sc_gather(h, neigh[: 7 * sA], 7, exact=True)
        GA = GA.reshape(sA, 7 * ic)
        GB = _sc_gather(h, neigh[7 * sA:], 7)
        GB = GB.reshape(GB.shape[0] // 7, 7 * ic)
        YA, SA = _mm(GA, W, b, n_true=sA)
        YB, SB = _mm(GB, W, b, n_true=n - sA)
        return _bn_lrelu2(YA, YB, SA, SB, gam, bet, n)
    G = _sc_gather(h, neigh, 7)
    G = G.reshape(G.shape[0] // 7, 7 * ic)
    Y, S = _mm(G, W, b, n_true=n)
    return _bn_lrelu(Y, S, gam, bet, n)


def _conv_block(h, neigh, p, n_true):
    h = _conv_layer(h, neigh, p['W1'], p['b1'], p['g1'], p['be1'], n_true)
    return _conv_layer(h, neigh, p['W2'], p['b2'], p['g2'], p['be2'], n_true)


def kernel(x, params, neighs, up_top, up_down):
    skips = [None] * _NLEV
    h = x
    for i in range(_NLEV):
        n_i = _LEVELS[i]
        if i > 0:
            pool_idx = neighs[i - 1][: 7 * n_i]
            P = _sc_gather(h, pool_idx, 7)
            h = _group_mean(P, 7)
        h = _conv_block(h, neighs[i], params['down'][i], n_i)
        skips[i] = h
    for i in range(_NLEV - 1):
        p = params['up'][i]
        lvl = _NLEV - 2 - i
        n_c = _LEVELS[lvl + 1]
        n_f = _LEVELS[lvl]
        oc = p['W2'].shape[0]
        y = _mm(h, p['Wu'], p['bu'])
        yv = y.reshape(y.shape[0] * 7, oc)
        ti2 = jnp.repeat(up_top[i], 2)
        idx2 = jnp.concatenate([ti2, up_down[i]])
        G2 = _sc_gather(yv, idx2, 2)
        u = _group_mean(G2, 2)
        npad = -(-n_f // 256) * 256
        table = jnp.concatenate([u[:npad], skips[lvl][:npad]], axis=1)
        ic = table.shape[1]
        W1r = p['W1'].reshape(oc, 7, ic).transpose(1, 0, 2).reshape(7 * oc, ic)
        A = _mm(table, W1r, jnp.zeros((7 * oc,), jnp.float32))
        A2 = A.reshape(A.shape[0] * 7, oc)
        idx7 = neighs[lvl] * 7 + jnp.tile(jnp.arange(7, dtype=jnp.int32), n_f)
        if n_f >= _SPLIT_MIN:
            sA = 256 * (n_f // 512)
            G7A = _sc_gather(A2, idx7[: 7 * sA], 7, exact=True)
            G7B = _sc_gather(A2, idx7[7 * sA:], 7)
            YA, SA = _group_sum_stats(G7A, 7, p['b1'], sA)
            YB, SB = _group_sum_stats(G7B, 7, p['b1'], n_f - sA)
            h1 = _bn_lrelu2(YA, YB, SA, SB, p['g1'], p['be1'], n_f)
        else:
            G7 = _sc_gather(A2, idx7, 7)
            Y, S = _group_sum_stats(G7, 7, p['b1'], n_f)
            h1 = _bn_lrelu(Y, S, p['g1'], p['be1'], n_f)
        h = _conv_layer(h1, neighs[lvl], p['W2'], p['b2'], p['g2'], p['be2'],
                        n_f)
    out = _mm(h, params['outW'], params['outb'])
    return out[: _LEVELS[0]]
